# Initial kernel scaffold; baseline (speedup 1.0000x reference)
import jax
import jax.numpy as jnp
from jax.experimental import pallas as pl


def kernel(x, embedding):
    raise NotImplementedError("write your pallas kernel here")


# speedup vs baseline: 2.9760x; 2.9760x over previous
import functools

import jax
import jax.numpy as jnp
from jax import lax
from jax.experimental import pallas as pl
from jax.experimental.pallas import tpu as pltpu
from jax.experimental.pallas import tpu_sc as plsc

_D = 128
_NC = 2
_NS = 16
_NW = _NC * _NS
_C = 128


@functools.partial(jax.jit, static_argnames=("n_rows",))
def _flat_gather(idx, table, n_rows):
    b_per_w = n_rows // _NW
    nch = b_per_w // _C
    mesh = plsc.VectorSubcoreMesh(core_axis_name="c", subcore_axis_name="s")

    @functools.partial(
        pl.kernel,
        out_type=jax.ShapeDtypeStruct((n_rows, _D), jnp.float32),
        mesh=mesh,
        scratch_types=[
            pltpu.VMEM((b_per_w,), jnp.int32),
            pltpu.VMEM((_C, _D), jnp.float32),
            pltpu.SemaphoreType.DMA,
        ],
    )
    def run(idx_hbm, table_hbm, out_hbm, idx_v, rows_v, sem):
        wid = lax.axis_index("s") * _NC + lax.axis_index("c")
        base = wid * b_per_w
        pltpu.sync_copy(idx_hbm.at[pl.ds(base, b_per_w)], idx_v)

        @pl.loop(0, nch)
        def chunk_loop(i):
            pltpu.async_copy(
                table_hbm.at[idx_v.at[pl.ds(i * _C, _C)]], rows_v, sem
            ).wait()
            pltpu.sync_copy(rows_v, out_hbm.at[pl.ds(base + i * _C, _C)])

    return run(idx, table)


def kernel(x, embedding):
    n = x.shape[0] * x.shape[1]
    idx = x.reshape(n).astype(jnp.int32)
    out = _flat_gather(idx, embedding, n)
    return out.reshape(x.shape + (embedding.shape[1],))


# speedup vs baseline: 3.1239x; 1.0497x over previous
import functools

import jax
import jax.numpy as jnp
from jax import lax
from jax.experimental import pallas as pl
from jax.experimental.pallas import tpu as pltpu
from jax.experimental.pallas import tpu_sc as plsc

_D = 128
_NC = 2
_NS = 16
_NW = _NC * _NS
_C = 128


@functools.partial(jax.jit, static_argnames=("n_rows",))
def _flat_gather(idx, table, n_rows):
    b_per_w = n_rows // _NW
    nch = b_per_w // _C
    mesh = plsc.VectorSubcoreMesh(core_axis_name="c", subcore_axis_name="s")

    @functools.partial(
        pl.kernel,
        out_type=jax.ShapeDtypeStruct((n_rows, _D), jnp.float32),
        mesh=mesh,
        scratch_types=[
            pltpu.VMEM((b_per_w,), jnp.int32),
            pltpu.VMEM((2, _C, _D), jnp.float32),
            pltpu.SemaphoreType.DMA,
            pltpu.SemaphoreType.DMA,
        ],
    )
    def run(idx_hbm, table_hbm, out_hbm, idx_v, rows_v, gsem, osem):
        wid = lax.axis_index("s") * _NC + lax.axis_index("c")
        base = wid * b_per_w
        pltpu.sync_copy(idx_hbm.at[pl.ds(base, b_per_w)], idx_v)

        def g_copy(ch, b):
            return pltpu.make_async_copy(
                table_hbm.at[idx_v.at[pl.ds(ch * _C, _C)]], rows_v.at[b], gsem
            )

        def o_copy(ch, b):
            return pltpu.make_async_copy(
                rows_v.at[b], out_hbm.at[pl.ds(base + ch * _C, _C)], osem
            )

        g_copy(0, 0).start()

        @pl.loop(0, nch, step=2)
        def chunk_loop(i):
            g_copy(i, 0).wait()

            @pl.when(i > 0)
            def _():
                o_copy(i - 1, 1).wait()

            g_copy(i + 1, 1).start()
            o_copy(i, 0).start()
            g_copy(i + 1, 1).wait()
            o_copy(i, 0).wait()

            @pl.when(i + 2 < nch)
            def _():
                g_copy(i + 2, 0).start()

            o_copy(i + 1, 1).start()

        o_copy(nch - 1, 1).wait()

    return run(idx, table)


def kernel(x, embedding):
    n = x.shape[0] * x.shape[1]
    idx = x.reshape(n).astype(jnp.int32)
    out = _flat_gather(idx, embedding, n)
    return out.reshape(x.shape + (embedding.shape[1],))


# speedup vs baseline: 3.3198x; 1.0627x over previous
import functools

import jax
import jax.numpy as jnp
from jax import lax
from jax.experimental import pallas as pl
from jax.experimental.pallas import tpu as pltpu
from jax.experimental.pallas import tpu_sc as plsc

_D = 128
_NC = 2
_NS = 16
_NW = _NC * _NS
_C = 400


@functools.partial(jax.jit, static_argnames=("n_rows",))
def _flat_gather(idx, table, n_rows):
    b_per_w = n_rows // _NW
    nch = b_per_w // _C
    mesh = plsc.VectorSubcoreMesh(core_axis_name="c", subcore_axis_name="s")

    @functools.partial(
        pl.kernel,
        out_type=jax.ShapeDtypeStruct((n_rows, _D), jnp.float32),
        mesh=mesh,
        scratch_types=[
            pltpu.VMEM((b_per_w,), jnp.int32),
            pltpu.VMEM((2, _C, _D), jnp.float32),
            pltpu.SemaphoreType.DMA,
            pltpu.SemaphoreType.DMA,
        ],
    )
    def run(idx_hbm, table_hbm, out_hbm, idx_v, rows_v, gsem, osem):
        wid = lax.axis_index("s") * _NC + lax.axis_index("c")
        base = wid * b_per_w
        pltpu.sync_copy(idx_hbm.at[pl.ds(base, b_per_w)], idx_v)

        def g_copy(ch, b):
            return pltpu.make_async_copy(
                table_hbm.at[idx_v.at[pl.ds(ch * _C, _C)]], rows_v.at[b], gsem
            )

        def o_copy(ch, b):
            return pltpu.make_async_copy(
                rows_v.at[b], out_hbm.at[pl.ds(base + ch * _C, _C)], osem
            )

        g_copy(0, 0).start()

        @pl.loop(0, nch, step=2)
        def chunk_loop(i):
            g_copy(i, 0).wait()

            @pl.when(i > 0)
            def _():
                o_copy(i - 1, 1).wait()

            g_copy(i + 1, 1).start()
            o_copy(i, 0).start()
            g_copy(i + 1, 1).wait()
            o_copy(i, 0).wait()

            @pl.when(i + 2 < nch)
            def _():
                g_copy(i + 2, 0).start()

            o_copy(i + 1, 1).start()

        o_copy(nch - 1, 1).wait()

    return run(idx, table)


def kernel(x, embedding):
    n = x.shape[0] * x.shape[1]
    idx = x.reshape(n).astype(jnp.int32)
    out = _flat_gather(idx, embedding, n)
    return out.reshape(x.shape + (embedding.shape[1],))


# speedup vs baseline: 5.8226x; 1.7539x over previous
import functools

import jax
import jax.numpy as jnp
from jax import lax
from jax.experimental import pallas as pl
from jax.experimental.pallas import tpu as pltpu
from jax.experimental.pallas import tpu_sc as plsc

_D = 128
_NC = 2
_NS = 16
_NW = _NC * _NS
_C = 400
_W = 50


@jax.jit
def _flat_gather(idx, table):
    n_rows = idx.shape[0]
    n_samp = n_rows // _W
    s_per_w = n_samp // _NW
    b_per_w = n_rows // _NW
    cs = _C // _W
    nch = b_per_w // _C
    mesh = plsc.VectorSubcoreMesh(core_axis_name="c", subcore_axis_name="s")

    @functools.partial(
        pl.kernel,
        out_type=jax.ShapeDtypeStruct((n_samp, _W, _D), jnp.float32),
        mesh=mesh,
        scratch_types=[
            pltpu.VMEM((b_per_w,), jnp.int32),
            pltpu.VMEM((2, _C, _D), jnp.float32),
            pltpu.SemaphoreType.DMA,
            pltpu.SemaphoreType.DMA,
        ],
    )
    def run(idx_hbm, table_hbm, out_hbm, idx_v, rows_v, gsem, osem):
        wid = lax.axis_index("s") * _NC + lax.axis_index("c")
        base = wid * b_per_w
        samp0 = wid * s_per_w
        pltpu.sync_copy(idx_hbm.at[pl.ds(base, b_per_w)], idx_v)

        def g_copy(ch, b):
            return pltpu.make_async_copy(
                table_hbm.at[idx_v.at[pl.ds(ch * _C, _C)]], rows_v.at[b], gsem
            )

        def o_copy(ch, b):
            return pltpu.make_async_copy(
                rows_v.at[b].reshape(cs, _W, _D),
                out_hbm.at[pl.ds(samp0 + ch * cs, cs)],
                osem,
            )

        g_copy(0, 0).start()

        @pl.loop(0, nch, step=2)
        def chunk_loop(i):
            g_copy(i, 0).wait()

            @pl.when(i > 0)
            def _():
                o_copy(i - 1, 1).wait()

            g_copy(i + 1, 1).start()
            o_copy(i, 0).start()
            g_copy(i + 1, 1).wait()
            o_copy(i, 0).wait()

            @pl.when(i + 2 < nch)
            def _():
                g_copy(i + 2, 0).start()

            o_copy(i + 1, 1).start()

        o_copy(nch - 1, 1).wait()

    return run(idx, table)


def kernel(x, embedding):
    idx = x.reshape(x.shape[0] * x.shape[1]).astype(jnp.int32)
    return _flat_gather(idx, embedding)
